# baseline (speedup 1.0000x reference)
import functools

import jax
import jax.numpy as jnp
from jax import lax
from jax.experimental import pallas as pl
from jax.experimental.pallas import tpu as pltpu
from jax.experimental.pallas import tpu_sc as plsc

VOCAB_CAT = 48
VOCAB_ATTR = 102
N_OBJ = 20
N_WORDS = 20

NC = 2
NS = 16
L = 16
NW = NC * NS


def _make_sc_call(B):
    assert B % NW == 0
    bpw = B // NW
    nb = 8
    assert bpw % (2 * nb) == 0
    npair = bpw // (2 * nb)
    rpw = bpw * N_OBJ
    mesh = plsc.VectorSubcoreMesh(core_axis_name="c", subcore_axis_name="s")

    @functools.partial(
        pl.kernel,
        mesh=mesh,
        compiler_params=pltpu.CompilerParams(
            needs_layout_passes=False, skip_device_barrier=True),
        out_type=[
            jax.ShapeDtypeStruct((B, N_OBJ, VOCAB_CAT), jnp.float32),
            jax.ShapeDtypeStruct((B, N_OBJ, VOCAB_ATTR), jnp.float32),
        ],
        scratch_types=[
            pltpu.VMEM((rpw,), jnp.int32),
            pltpu.VMEM((rpw * N_WORDS,), jnp.int32),
            pltpu.SemaphoreType.DMA,
            pltpu.SemaphoreType.DMA,
        ],
    )
    def sc_call(cat_hbm, attr_hbm, out1_hbm, out2_hbm,
                cat_v, attr_v, sem1, sem2):
        c = lax.axis_index("c")
        s = lax.axis_index("s")
        wid = s * NC + c
        row_base = wid * rpw

        pltpu.sync_copy(cat_hbm.at[pl.ds(row_base, rpw)], cat_v)
        pltpu.sync_copy(attr_hbm.at[pl.ds(row_base * N_WORDS, rpw * N_WORDS)],
                        attr_v)

        iota = lax.iota(jnp.int32, L)
        ones = jnp.full((L,), 1.0, jnp.float32)
        zeros = jnp.zeros((L,), jnp.float32)
        db_g = []
        nn_g = []
        for g in range(5):
            r = 16 * g + iota
            db_g.append(r // N_OBJ)
            nn_g.append(r % N_OBJ)

        def scoped(o1_a, o1_b, o2_a, o2_b):
            def do_block(o1p, o2p, blk, p):
                @pl.when(p > 0)
                def _():
                    pltpu.make_async_copy(
                        o1p, out1_hbm.at[pl.ds(0, nb)], sem1).wait()
                    pltpu.make_async_copy(
                        o2p, out2_hbm.at[pl.ds(0, nb)], sem2).wait()
                for b in range(nb):
                    for n in range(N_OBJ):
                        for j in range(VOCAB_CAT // L):
                            o1p[b, n, pl.ds(j * L, L)] = zeros
                        for off in (0, 16, 32, 48, 64, 80, VOCAB_ATTR - L):
                            o2p[b, n, pl.ds(off, L)] = zeros
                b0 = blk * nb
                for grp in range(nb * N_OBJ // L):
                    q, g = divmod(grp, 5)
                    bi = db_g[g] + q * 4
                    ni = nn_g[g]
                    r = b0 * N_OBJ + grp * L
                    catv = cat_v[pl.ds(r, L)]
                    plsc.store_scatter(o1p, [bi, ni, catv], ones)
                    ri_a = r * N_WORDS + iota * N_WORDS
                    for w in range(N_WORDS):
                        av = plsc.load_gather(attr_v, [ri_a + w])
                        plsc.store_scatter(o2p, [bi, ni, av], ones)
                gb0 = wid * bpw + b0
                pltpu.async_copy(o1p, out1_hbm.at[pl.ds(gb0, nb)], sem1)
                pltpu.async_copy(o2p, out2_hbm.at[pl.ds(gb0, nb)], sem2)

            def pair(p, carry):
                do_block(o1_a, o2_a, 2 * p, p)
                do_block(o1_b, o2_b, 2 * p + 1, p)
                return carry

            lax.fori_loop(0, npair, pair, 0)

            for _ in range(2):
                pltpu.make_async_copy(
                    o1_a, out1_hbm.at[pl.ds(0, nb)], sem1).wait()
                pltpu.make_async_copy(
                    o2_a, out2_hbm.at[pl.ds(0, nb)], sem2).wait()

        pl.run_scoped(
            scoped,
            pltpu.VMEM((nb, N_OBJ, VOCAB_CAT), jnp.float32),
            pltpu.VMEM((nb, N_OBJ, VOCAB_CAT), jnp.float32),
            pltpu.VMEM((nb, N_OBJ, VOCAB_ATTR), jnp.float32),
            pltpu.VMEM((nb, N_OBJ, VOCAB_ATTR), jnp.float32),
        )

    return sc_call


@jax.jit
def kernel(category, attributes):
    B, N, _ = category.shape
    cat_flat = category.reshape(B * N)
    attr_flat = attributes.reshape(B * N * N_WORDS)
    return tuple(_make_sc_call(B)(cat_flat, attr_flat))


# speedup vs baseline: 1.5258x; 1.5258x over previous
import functools

import jax
import jax.numpy as jnp
from jax import lax
from jax.experimental import pallas as pl
from jax.experimental.pallas import tpu as pltpu

VOCAB_CAT = 48
VOCAB_ATTR = 102
N_OBJ = 20
N_WORDS = 20


def _tc_body(cat_ref, attr_ref, out1_ref, out2_ref):
    bb = cat_ref.shape[0]
    rows = bb * N_OBJ

    iota1 = lax.broadcasted_iota(jnp.int32, (bb, N_OBJ, VOCAB_CAT), 2)
    out1_ref[...] = (cat_ref[...] == iota1).astype(jnp.float32)

    a = attr_ref[...].reshape(rows, N_WORDS)
    a = jnp.transpose(a, (1, 0))
    bit = jnp.left_shift(jnp.int32(1), a & 31)
    hi = jnp.right_shift(a, 5)

    def or_reduce0(x):
        s = x.shape[0]
        while s > 1:
            h = (s + 1) // 2
            x = x[:h] | x[s - h:s]
            s = h
        return x

    words = []
    for k in range(4):
        contrib = jnp.where(hi == k, bit, 0)
        words.append(or_reduce0(contrib))
    mask4 = jnp.concatenate(words, axis=0)
    mask4 = jnp.transpose(mask4, (1, 0))
    mask4 = mask4.reshape(bb, N_OBJ, 4)

    iota2 = lax.broadcasted_iota(jnp.int32, (bb, N_OBJ, VOCAB_ATTR), 2)
    sel = jnp.take_along_axis(mask4, jnp.right_shift(iota2, 5), axis=2)
    bits = jnp.right_shift(sel, iota2 & 31) & 1
    out2_ref[...] = bits.astype(jnp.float32)


@jax.jit
def kernel(category, attributes):
    B, N, _ = category.shape
    bb = 256
    grid = (B // bb,)
    return pl.pallas_call(
        _tc_body,
        grid=grid,
        in_specs=[
            pl.BlockSpec((bb, N, 1), lambda i: (i, 0, 0)),
            pl.BlockSpec((bb, N, N_WORDS), lambda i: (i, 0, 0)),
        ],
        out_specs=[
            pl.BlockSpec((bb, N, VOCAB_CAT), lambda i: (i, 0, 0)),
            pl.BlockSpec((bb, N, VOCAB_ATTR), lambda i: (i, 0, 0)),
        ],
        out_shape=[
            jax.ShapeDtypeStruct((B, N, VOCAB_CAT), jnp.float32),
            jax.ShapeDtypeStruct((B, N, VOCAB_ATTR), jnp.float32),
        ],
    )(category, attributes)
